# Initial kernel scaffold; baseline (speedup 1.0000x reference)
import functools

import jax
import jax.numpy as jnp
from jax import lax
from jax.experimental import pallas as pl
from jax.experimental.pallas import tpu as pltpu
from jax.experimental.pallas import tpu_sc as plsc

N = 10000
E = 320000
D_IN = 128
H = 16
C = 32

NC = 2
NS = 16
NW = NC * NS
BG = 125
G = E // (NW * BG)
N_ACC = 10240
RPT = N_ACC // NS

_mesh = plsc.VectorSubcoreMesh(core_axis_name="c", subcore_axis_name="s")


@functools.partial(
    pl.kernel,
    out_type=jax.ShapeDtypeStruct((NC, N_ACC), jnp.float32),
    mesh=_mesh,
    scratch_types=[
        pltpu.VMEM((G, BG), jnp.int32),
        pltpu.VMEM((RPT,), jnp.float32),
        pltpu.VMEM((128,), jnp.float32),
        pltpu.VMEM_SHARED((N_ACC,), jnp.float32),
    ],
)
def _deg_kernel(col_hbm, out_hbm, col_v, zbuf, ones_v, acc):
    c = lax.axis_index("c")
    s = lax.axis_index("s")
    wid = c * NS + s

    def zi(i, _):
        zbuf[pl.ds(i * 16, 16)] = jnp.zeros((16,), jnp.float32)
        return 0

    lax.fori_loop(0, RPT // 16, zi, 0)
    for t in range(8):
        ones_v[pl.ds(t * 16, 16)] = jnp.ones((16,), jnp.float32)
    pltpu.sync_copy(zbuf, acc.at[pl.ds(s * RPT, RPT)])
    pltpu.sync_copy(col_hbm.at[wid], col_v)
    plsc.subcore_barrier()

    def body(j, _):
        pltpu.sync_copy(ones_v.at[pl.ds(0, BG)], acc.at[col_v.at[j]], add=True)
        return 0

    lax.fori_loop(0, G, body, 0)
    plsc.subcore_barrier()
    pltpu.sync_copy(acc.at[pl.ds(s * RPT, RPT)],
                    out_hbm.at[c, pl.ds(s * RPT, RPT)])


def _make_agg(D):
    @functools.partial(
        pl.kernel,
        out_type=jax.ShapeDtypeStruct((NC, N_ACC, D), jnp.float32),
        mesh=_mesh,
        scratch_types=[
            pltpu.VMEM((G, BG), jnp.int32),
            pltpu.VMEM((G, BG), jnp.int32),
            pltpu.VMEM((2, BG, D), jnp.float32),
            pltpu.VMEM((RPT, D), jnp.float32),
            pltpu.VMEM_SHARED((N_ACC, D), jnp.float32),
            pltpu.SemaphoreType.DMA((2,)),
        ],
    )
    def agg(g_hbm, row_hbm, col_hbm, out_hbm, row_v, col_v, ring, zbuf, acc,
            sems):
        c = lax.axis_index("c")
        s = lax.axis_index("s")
        wid = c * NS + s

        def zrow(i, _):
            for t in range(D // 16):
                zbuf[i, pl.ds(t * 16, 16)] = jnp.zeros((16,), jnp.float32)
            return 0

        lax.fori_loop(0, RPT, zrow, 0)
        pltpu.sync_copy(zbuf, acc.at[pl.ds(s * RPT, RPT)])
        pltpu.sync_copy(row_hbm.at[wid], row_v)
        pltpu.sync_copy(col_hbm.at[wid], col_v)
        plsc.subcore_barrier()

        for b in range(2):
            pltpu.async_copy(g_hbm.at[row_v.at[b]], ring.at[b], sems.at[b])

        def step(t, _):
            for b in range(2):
                j = t * 2 + b
                pltpu.make_async_copy(g_hbm.at[row_v.at[j]], ring.at[b],
                                      sems.at[b]).wait()
                pltpu.sync_copy(ring.at[b], acc.at[col_v.at[j]], add=True)
                nxt = j + 2

                @pl.when(nxt < G)
                def _():
                    pltpu.async_copy(g_hbm.at[row_v.at[nxt]], ring.at[b],
                                     sems.at[b])
            return 0

        lax.fori_loop(0, G // 2, step, 0)
        plsc.subcore_barrier()
        pltpu.sync_copy(acc.at[pl.ds(s * RPT, RPT)],
                        out_hbm.at[c, pl.ds(s * RPT, RPT), :])

    return agg


_agg_h = _make_agg(H)
_agg_c = _make_agg(C)


def _tc1_body(deg_ref, x_ref, w1_ref, dinv_ref, g1_ref):
    deg = deg_ref[0] + deg_ref[1] + 1.0
    dinv = lax.rsqrt(deg)
    dinv_ref[...] = dinv
    h = jnp.dot(x_ref[...], w1_ref[...], preferred_element_type=jnp.float32)
    g1_ref[...] = h * dinv


def _tc2_body(acc_ref, g1_ref, dinv_ref, b1_ref, w2_ref, g2_ref):
    dinv = dinv_ref[...]
    pre = (acc_ref[0] + acc_ref[1] + g1_ref[...]) * dinv + b1_ref[...]
    a = jnp.maximum(pre, 0.0)
    h2 = jnp.dot(a, w2_ref[...], preferred_element_type=jnp.float32)
    g2_ref[...] = h2 * dinv


def _tc3_body(acc_ref, g2_ref, dinv_ref, b2_ref, out_ref):
    o = (acc_ref[0] + acc_ref[1] + g2_ref[...]) * dinv_ref[...] + b2_ref[...]
    m = jnp.max(o, axis=1, keepdims=True)
    lse = jnp.log(jnp.sum(jnp.exp(o - m), axis=1, keepdims=True)) + m
    out_ref[...] = o - lse


_tc1 = pl.pallas_call(
    _tc1_body,
    out_shape=(jax.ShapeDtypeStruct((N_ACC, 1), jnp.float32),
               jax.ShapeDtypeStruct((N_ACC, H), jnp.float32)))
_tc2 = pl.pallas_call(
    _tc2_body,
    out_shape=jax.ShapeDtypeStruct((N_ACC, C), jnp.float32))
_tc3 = pl.pallas_call(
    _tc3_body,
    out_shape=jax.ShapeDtypeStruct((N_ACC, C), jnp.float32))


def kernel(x, edge_index, W1, b1, W2, b2):
    ei = edge_index.astype(jnp.int32)
    row = ei[0].reshape(NW, G, BG)
    col = ei[1].reshape(NW, G, BG)
    xp = jnp.pad(x, ((0, N_ACC - N), (0, 0)))

    deg = _deg_kernel(col)
    dinv, g1 = _tc1(deg.reshape(NC, N_ACC, 1), xp, W1)
    acc1 = _agg_h(g1, row, col)
    g2 = _tc2(acc1, g1, dinv, b1.reshape(1, H), W2)
    acc2 = _agg_c(g2, row, col)
    out = _tc3(acc2, g2, dinv, b2.reshape(1, C))
    return out[:N]


# speedup vs baseline: 43.9724x; 43.9724x over previous
import functools

import jax
import jax.numpy as jnp
from jax import lax
from jax.experimental import pallas as pl
from jax.experimental.pallas import tpu as pltpu
from jax.experimental.pallas import tpu_sc as plsc

N = 10000
E = 320000
D_IN = 128
H = 16
C = 32

NC = 2
NS = 16
NW = NC * NS
BG = 125
G = E // (NW * BG)
N_ACC = 10240
RPT = N_ACC // NS

_mesh = plsc.VectorSubcoreMesh(core_axis_name="c", subcore_axis_name="s")
_sc_params = pltpu.CompilerParams(use_tc_tiling_on_sc=False)


@functools.partial(
    pl.kernel,
    out_type=jax.ShapeDtypeStruct((NC, N_ACC), jnp.float32),
    mesh=_mesh,
    compiler_params=_sc_params,
    scratch_types=[
        pltpu.VMEM((G, BG), jnp.int32),
        pltpu.VMEM((RPT,), jnp.float32),
        pltpu.VMEM((128,), jnp.float32),
        pltpu.VMEM_SHARED((N_ACC,), jnp.float32),
    ],
)
def _deg_kernel(col_hbm, out_hbm, col_v, zbuf, ones_v, acc):
    c = lax.axis_index("c")
    s = lax.axis_index("s")
    wid = c * NS + s

    def zi(i, _):
        zbuf[pl.ds(i * 16, 16)] = jnp.zeros((16,), jnp.float32)
        return 0

    lax.fori_loop(0, RPT // 16, zi, 0)
    for t in range(8):
        ones_v[pl.ds(t * 16, 16)] = jnp.ones((16,), jnp.float32)
    pltpu.sync_copy(zbuf, acc.at[pl.ds(s * RPT, RPT)])
    pltpu.sync_copy(col_hbm.at[wid], col_v)
    plsc.subcore_barrier()

    def body(j, _):
        pltpu.sync_copy(ones_v.at[pl.ds(0, BG)], acc.at[col_v.at[j]], add=True)
        return 0

    lax.fori_loop(0, G, body, 0)
    plsc.subcore_barrier()
    pltpu.sync_copy(acc.at[pl.ds(s * RPT, RPT)],
                    out_hbm.at[c, pl.ds(s * RPT, RPT)])


def _make_agg(D):
    @functools.partial(
        pl.kernel,
        out_type=jax.ShapeDtypeStruct((NC, N_ACC, D), jnp.float32),
        mesh=_mesh,
        compiler_params=_sc_params,
        scratch_types=[
            pltpu.VMEM((G, BG), jnp.int32),
            pltpu.VMEM((G, BG), jnp.int32),
            pltpu.VMEM((2, BG, D), jnp.float32),
            pltpu.VMEM((RPT, D), jnp.float32),
            pltpu.VMEM_SHARED((N_ACC, D), jnp.float32),
            pltpu.SemaphoreType.DMA((2,)),
        ],
    )
    def agg(g_hbm, row_hbm, col_hbm, out_hbm, row_v, col_v, ring, zbuf, acc,
            sems):
        c = lax.axis_index("c")
        s = lax.axis_index("s")
        wid = c * NS + s

        def zrow(i, _):
            for t in range(D // 16):
                zbuf[i, pl.ds(t * 16, 16)] = jnp.zeros((16,), jnp.float32)
            return 0

        lax.fori_loop(0, RPT, zrow, 0)
        pltpu.sync_copy(zbuf, acc.at[pl.ds(s * RPT, RPT)])
        pltpu.sync_copy(row_hbm.at[wid], row_v)
        pltpu.sync_copy(col_hbm.at[wid], col_v)
        plsc.subcore_barrier()

        for b in range(2):
            pltpu.async_copy(g_hbm.at[row_v.at[b]], ring.at[b], sems.at[b])

        def step(t, _):
            for b in range(2):
                j = t * 2 + b
                pltpu.make_async_copy(g_hbm.at[row_v.at[j]], ring.at[b],
                                      sems.at[b]).wait()
                pltpu.sync_copy(ring.at[b], acc.at[col_v.at[j]], add=True)
                nxt = j + 2

                @pl.when(nxt < G)
                def _():
                    pltpu.async_copy(g_hbm.at[row_v.at[nxt]], ring.at[b],
                                     sems.at[b])
            return 0

        lax.fori_loop(0, G // 2, step, 0)
        plsc.subcore_barrier()
        pltpu.sync_copy(acc.at[pl.ds(s * RPT, RPT)],
                        out_hbm.at[c, pl.ds(s * RPT, RPT), :])

    return agg


_agg_h = _make_agg(H)
_agg_c = _make_agg(C)


def _tc1_body(deg_ref, x_ref, w1_ref, dinv_ref, g1_ref):
    deg = deg_ref[0] + deg_ref[1] + 1.0
    dinv = lax.rsqrt(deg)
    dinv_ref[...] = dinv
    h = jnp.dot(x_ref[...], w1_ref[...], preferred_element_type=jnp.float32)
    g1_ref[...] = h * dinv


def _tc2_body(acc_ref, g1_ref, dinv_ref, b1_ref, w2_ref, g2_ref):
    dinv = dinv_ref[...]
    pre = (acc_ref[0] + acc_ref[1] + g1_ref[...]) * dinv + b1_ref[...]
    a = jnp.maximum(pre, 0.0)
    h2 = jnp.dot(a, w2_ref[...], preferred_element_type=jnp.float32)
    g2_ref[...] = h2 * dinv


def _tc3_body(acc_ref, g2_ref, dinv_ref, b2_ref, out_ref):
    o = (acc_ref[0] + acc_ref[1] + g2_ref[...]) * dinv_ref[...] + b2_ref[...]
    m = jnp.max(o, axis=1, keepdims=True)
    lse = jnp.log(jnp.sum(jnp.exp(o - m), axis=1, keepdims=True)) + m
    out_ref[...] = o - lse


_tc1 = pl.pallas_call(
    _tc1_body,
    out_shape=(jax.ShapeDtypeStruct((N_ACC, 1), jnp.float32),
               jax.ShapeDtypeStruct((N_ACC, H), jnp.float32)))
_tc2 = pl.pallas_call(
    _tc2_body,
    out_shape=jax.ShapeDtypeStruct((N_ACC, C), jnp.float32))
_tc3 = pl.pallas_call(
    _tc3_body,
    out_shape=jax.ShapeDtypeStruct((N_ACC, C), jnp.float32))


def kernel(x, edge_index, W1, b1, W2, b2):
    ei = edge_index.astype(jnp.int32)
    row = ei[0].reshape(NW, G, BG)
    col = ei[1].reshape(NW, G, BG)
    xp = jnp.pad(x, ((0, N_ACC - N), (0, 0)))

    deg = _deg_kernel(col)
    dinv, g1 = _tc1(deg.reshape(NC, N_ACC, 1), xp, W1)
    acc1 = _agg_h(g1, row, col)
    g2 = _tc2(acc1, g1, dinv, b1.reshape(1, H), W2)
    acc2 = _agg_c(g2, row, col)
    out = _tc3(acc2, g2, dinv, b2.reshape(1, C))
    return out[:N]
